# baseline (speedup 1.0000x reference)
import functools

import jax
import jax.numpy as jnp
from jax import lax
from jax.experimental import pallas as pl
from jax.experimental.pallas import tpu as pltpu
from jax.experimental.pallas import tpu_sc as plsc

_N = 1048576
_C = 128
_NBINS = 15
_NC = 2
_NS = 16
_NW = _NC * _NS
_CHUNK = 256
_GROUPS = _CHUNK // 16

_SC_ROWS = 557056
assert _SC_ROWS % (_NW * _CHUNK * 2) == 0
_SC_NCHUNK = _SC_ROWS // _NW // _CHUNK
_ROWS_PER_W = _SC_ROWS // _NW

_TC_R = 2048
_TC_NB = (_N - _SC_ROWS) // _TC_R
_TC_BLK0 = _SC_ROWS // _TC_R
assert _SC_ROWS + _TC_NB * _TC_R == _N and _SC_ROWS % _TC_R == 0

def _sc_body(sm_hbm, lbl_hbm, out_hbm,
             rows0, rows1, lbl0, lbl1, acc,
             sem_r0, sem_r1, sem_l0, sem_l1):
    wid = lax.axis_index("s") * _NC + lax.axis_index("c")
    base = wid * _ROWS_PER_W

    rows_bufs = (rows0, rows1)
    lbl_bufs = (lbl0, lbl1)
    sem_r = (sem_r0, sem_r1)
    sem_l = (sem_l0, sem_l1)

    lanes = lax.iota(jnp.int32, 16)
    zero16 = jnp.zeros((16,), jnp.float32)

    for t in range(48):
        acc[pl.ds(t * 16, 16)] = zero16

    def start(cur, b):
        r0 = base + cur * _CHUNK
        pltpu.async_copy(sm_hbm.at[pl.ds(r0 * _C, _CHUNK * _C)], rows_bufs[b],
                         sem_r[b])
        pltpu.async_copy(lbl_hbm.at[pl.ds(r0, _CHUNK)], lbl_bufs[b], sem_l[b])

    def wait(cur, b):
        r0 = base + cur * _CHUNK
        pltpu.make_async_copy(
            sm_hbm.at[pl.ds(r0 * _C, _CHUNK * _C)], rows_bufs[b], sem_r[b]).wait()
        pltpu.make_async_copy(
            lbl_hbm.at[pl.ds(r0, _CHUNK)], lbl_bufs[b], sem_l[b]).wait()

    nblk = _C // 16
    idx_pat = [lanes * _C + ((lanes + u) & 15) for u in range(16)]

    def compute(b):
        rows = rows_bufs[b]
        lblv = lbl_bufs[b]

        def gbody(g, _):
            gbase = g * (16 * _C)

            m_blks = []
            for bb in range(nblk):
                slb = rows.at[pl.ds(gbase + bb * 16, 15 * _C + 16)]
                mb = jnp.full((16,), -1.0, jnp.float32)
                for u in range(16):
                    v = plsc.load_gather(slb, [idx_pat[u]])
                    mb = jnp.maximum(mb, v)
                m_blks.append(mb)

            t1 = [jnp.maximum(m_blks[2 * i], m_blks[2 * i + 1]) for i in range(4)]
            t2 = [jnp.maximum(t1[0], t1[1]), jnp.maximum(t1[2], t1[3])]
            m = jnp.maximum(t2[0], t2[1])

            amb = jnp.zeros((16,), jnp.int32)
            for bb in range(nblk - 1, 0, -1):
                amb = jnp.where(m_blks[bb] == m,
                                jnp.full((16,), bb * 16, jnp.int32), amb)

            sl = rows.at[pl.ds(gbase, 16 * _C)]
            am = amb
            for u in range(15, -1, -1):
                idxfull = idx_pat[u] + amb
                v = plsc.load_gather(sl, [idxfull])
                am = jnp.where(v == m, idxfull & 127, am)

            lbl = lblv[pl.ds(g * 16, 16)]
            accv = jnp.where(am == lbl, 1.0, 0.0).astype(jnp.float32)

            bini = (m * jnp.float32(_NBINS)).astype(jnp.int32)
            binv = jnp.minimum(bini, _NBINS - 1)
            validf = jnp.where(m > 0.0, 1.0, 0.0).astype(jnp.float32)

            slot = binv * 16 + lanes
            plsc.addupdate_scatter(acc, [slot], validf)
            plsc.addupdate_scatter(acc, [slot + 256], m * validf)
            plsc.addupdate_scatter(acc, [slot + 512], accv * validf)
            return 0

        lax.fori_loop(0, _GROUPS, gbody, 0)

    start(0, 0)
    start(1, 1)

    def outer(it, _):
        i = it * 2
        for b in range(2):
            cur = i + b
            wait(cur, b)
            compute(b)

            @pl.when(cur + 2 < _SC_NCHUNK)
            def _prefetch():
                start(cur + 2, b)

        return 0

    lax.fori_loop(0, _SC_NCHUNK // 2, outer, 0)

    pltpu.sync_copy(acc, out_hbm.at[wid])


_sc_partials = functools.partial(
    pl.kernel,
    out_type=jax.ShapeDtypeStruct((_NW, 768), jnp.float32),
    mesh=plsc.VectorSubcoreMesh(core_axis_name="c", subcore_axis_name="s"),
    compiler_params=pltpu.CompilerParams(needs_layout_passes=False),
    scratch_types=[
        pltpu.VMEM((_CHUNK * _C,), jnp.float32),
        pltpu.VMEM((_CHUNK * _C,), jnp.float32),
        pltpu.VMEM((_CHUNK,), jnp.int32),
        pltpu.VMEM((_CHUNK,), jnp.int32),
        pltpu.VMEM((768,), jnp.float32),
        pltpu.SemaphoreType.DMA,
        pltpu.SemaphoreType.DMA,
        pltpu.SemaphoreType.DMA,
        pltpu.SemaphoreType.DMA,
    ],
)(_sc_body)


def _tc_body(lbl_ref, sm_ref, out_ref):
    i = pl.program_id(0)

    @pl.when(i == 0)
    def _init():
        out_ref[...] = jnp.zeros((8, 128), jnp.float32)

    xt = sm_ref[...].T
    conf = jnp.max(xt, axis=0, keepdims=True)
    iot = lax.broadcasted_iota(jnp.int32, (_C, _TC_R), 0)
    cand = jnp.where(xt == conf, iot, _C)
    pred = jnp.min(cand, axis=0, keepdims=True)

    lbl = lbl_ref[0]
    accv = jnp.where(pred == lbl, 1.0, 0.0).astype(jnp.float32)

    bini = (conf * jnp.float32(_NBINS)).astype(jnp.int32)
    t = jnp.where(conf > 0.0, jnp.minimum(bini, _NBINS - 1) + 1, 0)

    onehot = jnp.where(
        lax.broadcasted_iota(jnp.int32, (16, _TC_R), 0) == t, 1.0, 0.0
    ).astype(jnp.float32)
    vals = jnp.concatenate(
        [jnp.ones((1, _TC_R), jnp.float32), conf, accv], axis=0)
    stats = lax.dot_general(vals, onehot, (((1,), (1,)), ((), ())),
                            preferred_element_type=jnp.float32)
    out_ref[0:3, 0:16] = out_ref[0:3, 0:16] + stats


_tc_stats = pl.pallas_call(
    _tc_body,
    grid=(_TC_NB,),
    in_specs=[
        pl.BlockSpec((1, 1, _TC_R), lambda i: (_TC_BLK0 + i, 0, 0)),
        pl.BlockSpec((_TC_R, _C), lambda i: (_TC_BLK0 + i, 0)),
    ],
    out_specs=pl.BlockSpec((8, 128), lambda i: (0, 0)),
    out_shape=jax.ShapeDtypeStruct((8, 128), jnp.float32),
)


def kernel(softmaxes, labels):
    parts_sc = _sc_partials(softmaxes.reshape(-1), labels)
    parts_tc = _tc_stats(labels.reshape(-1, 1, _TC_R), softmaxes)
    s = jnp.sum(parts_sc, axis=0).reshape(48, 16).sum(axis=1)
    cnt = s[0:_NBINS] + parts_tc[0, 1:16]
    sum_conf = s[16:16 + _NBINS] + parts_tc[1, 1:16]
    sum_acc = s[32:32 + _NBINS] + parts_tc[2, 1:16]
    prop = cnt / _N
    safe = jnp.maximum(cnt, 1.0)
    contrib = jnp.abs(sum_conf / safe - sum_acc / safe) * prop
    ece = jnp.sum(jnp.where(prop > 0.0, contrib, 0.0))
    return ece.reshape(1)


# speedup vs baseline: 1.1431x; 1.1431x over previous
import functools

import jax
import jax.numpy as jnp
from jax import lax
from jax.experimental import pallas as pl
from jax.experimental.pallas import tpu as pltpu
from jax.experimental.pallas import tpu_sc as plsc

_N = 1048576
_C = 128
_NBINS = 15
_NC = 2
_NS = 16
_NW = _NC * _NS
_CHUNK = 256
_GROUPS = _CHUNK // 16

_SC_ROWS = 638976
assert _SC_ROWS % (_NW * _CHUNK * 2) == 0
_SC_NCHUNK = _SC_ROWS // _NW // _CHUNK
_ROWS_PER_W = _SC_ROWS // _NW

_TC_R = 2048
_TC_NB = (_N - _SC_ROWS) // _TC_R
_TC_BLK0 = _SC_ROWS // _TC_R
assert _SC_ROWS + _TC_NB * _TC_R == _N and _SC_ROWS % _TC_R == 0

def _sc_body(sm_hbm, lbl_hbm, out_hbm,
             rows0, rows1, lbl0, lbl1, acc,
             sem_r0, sem_r1, sem_l0, sem_l1):
    wid = lax.axis_index("s") * _NC + lax.axis_index("c")
    base = wid * _ROWS_PER_W

    rows_bufs = (rows0, rows1)
    lbl_bufs = (lbl0, lbl1)
    sem_r = (sem_r0, sem_r1)
    sem_l = (sem_l0, sem_l1)

    lanes = lax.iota(jnp.int32, 16)
    zero16 = jnp.zeros((16,), jnp.float32)

    for t in range(48):
        acc[pl.ds(t * 16, 16)] = zero16

    def start(cur, b):
        r0 = base + cur * _CHUNK
        pltpu.async_copy(sm_hbm.at[pl.ds(r0 * _C, _CHUNK * _C)], rows_bufs[b],
                         sem_r[b])
        pltpu.async_copy(lbl_hbm.at[pl.ds(r0, _CHUNK)], lbl_bufs[b], sem_l[b])

    def wait(cur, b):
        r0 = base + cur * _CHUNK
        pltpu.make_async_copy(
            sm_hbm.at[pl.ds(r0 * _C, _CHUNK * _C)], rows_bufs[b], sem_r[b]).wait()
        pltpu.make_async_copy(
            lbl_hbm.at[pl.ds(r0, _CHUNK)], lbl_bufs[b], sem_l[b]).wait()

    nblk = _C // 16
    idx_pat = [lanes * _C + ((lanes + u) & 15) for u in range(16)]

    def compute(b):
        rows = rows_bufs[b]
        lblv = lbl_bufs[b]

        def gbody(g, _):
            gbase = g * (16 * _C)

            m_blks = []
            for bb in range(nblk):
                slb = rows.at[pl.ds(gbase + bb * 16, 15 * _C + 16)]
                mb = jnp.full((16,), -1.0, jnp.float32)
                for u in range(16):
                    v = plsc.load_gather(slb, [idx_pat[u]])
                    mb = jnp.maximum(mb, v)
                m_blks.append(mb)

            t1 = [jnp.maximum(m_blks[2 * i], m_blks[2 * i + 1]) for i in range(4)]
            t2 = [jnp.maximum(t1[0], t1[1]), jnp.maximum(t1[2], t1[3])]
            m = jnp.maximum(t2[0], t2[1])

            amb = jnp.zeros((16,), jnp.int32)
            for bb in range(nblk - 1, 0, -1):
                amb = jnp.where(m_blks[bb] == m,
                                jnp.full((16,), bb * 16, jnp.int32), amb)

            sl = rows.at[pl.ds(gbase, 16 * _C)]
            am = amb
            for u in range(15, -1, -1):
                idxfull = idx_pat[u] + amb
                v = plsc.load_gather(sl, [idxfull])
                am = jnp.where(v == m, idxfull & 127, am)

            lbl = lblv[pl.ds(g * 16, 16)]
            accv = jnp.where(am == lbl, 1.0, 0.0).astype(jnp.float32)

            bini = (m * jnp.float32(_NBINS)).astype(jnp.int32)
            binv = jnp.minimum(bini, _NBINS - 1)
            validf = jnp.where(m > 0.0, 1.0, 0.0).astype(jnp.float32)

            slot = binv * 16 + lanes
            plsc.addupdate_scatter(acc, [slot], validf)
            plsc.addupdate_scatter(acc, [slot + 256], m * validf)
            plsc.addupdate_scatter(acc, [slot + 512], accv * validf)
            return 0

        lax.fori_loop(0, _GROUPS, gbody, 0)

    start(0, 0)
    start(1, 1)

    def outer(it, _):
        i = it * 2
        for b in range(2):
            cur = i + b
            wait(cur, b)
            compute(b)

            @pl.when(cur + 2 < _SC_NCHUNK)
            def _prefetch():
                start(cur + 2, b)

        return 0

    lax.fori_loop(0, _SC_NCHUNK // 2, outer, 0)

    pltpu.sync_copy(acc, out_hbm.at[wid])


_sc_partials = functools.partial(
    pl.kernel,
    out_type=jax.ShapeDtypeStruct((_NW, 768), jnp.float32),
    mesh=plsc.VectorSubcoreMesh(core_axis_name="c", subcore_axis_name="s"),
    compiler_params=pltpu.CompilerParams(needs_layout_passes=False),
    scratch_types=[
        pltpu.VMEM((_CHUNK * _C,), jnp.float32),
        pltpu.VMEM((_CHUNK * _C,), jnp.float32),
        pltpu.VMEM((_CHUNK,), jnp.int32),
        pltpu.VMEM((_CHUNK,), jnp.int32),
        pltpu.VMEM((768,), jnp.float32),
        pltpu.SemaphoreType.DMA,
        pltpu.SemaphoreType.DMA,
        pltpu.SemaphoreType.DMA,
        pltpu.SemaphoreType.DMA,
    ],
)(_sc_body)


def _tc_body(lbl_ref, sm_ref, out_ref):
    i = pl.program_id(0)

    @pl.when(i == 0)
    def _init():
        out_ref[...] = jnp.zeros((8, 128), jnp.float32)

    xt = sm_ref[...].T
    conf = jnp.max(xt, axis=0, keepdims=True)
    iot = lax.broadcasted_iota(jnp.int32, (_C, _TC_R), 0)
    cand = jnp.where(xt == conf, iot, _C)
    pred = jnp.min(cand, axis=0, keepdims=True)

    lbl = lbl_ref[0]
    accv = jnp.where(pred == lbl, 1.0, 0.0).astype(jnp.float32)

    bini = (conf * jnp.float32(_NBINS)).astype(jnp.int32)
    t = jnp.where(conf > 0.0, jnp.minimum(bini, _NBINS - 1) + 1, 0)

    onehot = jnp.where(
        lax.broadcasted_iota(jnp.int32, (16, _TC_R), 0) == t, 1.0, 0.0
    ).astype(jnp.float32)
    vals = jnp.concatenate(
        [jnp.ones((1, _TC_R), jnp.float32), conf, accv], axis=0)
    stats = lax.dot_general(vals, onehot, (((1,), (1,)), ((), ())),
                            preferred_element_type=jnp.float32)
    out_ref[0:3, 0:16] = out_ref[0:3, 0:16] + stats


_tc_stats = pl.pallas_call(
    _tc_body,
    grid=(_TC_NB,),
    in_specs=[
        pl.BlockSpec((1, 1, _TC_R), lambda i: (_TC_BLK0 + i, 0, 0)),
        pl.BlockSpec((_TC_R, _C), lambda i: (_TC_BLK0 + i, 0)),
    ],
    out_specs=pl.BlockSpec((8, 128), lambda i: (0, 0)),
    out_shape=jax.ShapeDtypeStruct((8, 128), jnp.float32),
)


def kernel(softmaxes, labels):
    parts_sc = _sc_partials(softmaxes.reshape(-1), labels)
    parts_tc = _tc_stats(labels.reshape(-1, 1, _TC_R), softmaxes)
    s = jnp.sum(parts_sc, axis=0).reshape(48, 16).sum(axis=1)
    cnt = s[0:_NBINS] + parts_tc[0, 1:16]
    sum_conf = s[16:16 + _NBINS] + parts_tc[1, 1:16]
    sum_acc = s[32:32 + _NBINS] + parts_tc[2, 1:16]
    prop = cnt / _N
    safe = jnp.maximum(cnt, 1.0)
    contrib = jnp.abs(sum_conf / safe - sum_acc / safe) * prop
    ece = jnp.sum(jnp.where(prop > 0.0, contrib, 0.0))
    return ece.reshape(1)


# speedup vs baseline: 1.2386x; 1.0836x over previous
import functools

import jax
import jax.numpy as jnp
from jax import lax
from jax.experimental import pallas as pl
from jax.experimental.pallas import tpu as pltpu
from jax.experimental.pallas import tpu_sc as plsc

_N = 1048576
_C = 128
_NBINS = 15
_NC = 2
_NS = 16
_NW = _NC * _NS
_CHUNK = 256
_GROUPS = _CHUNK // 16

_SC_ROWS = 638976
assert _SC_ROWS % (_NW * _CHUNK * 2) == 0
_SC_NCHUNK = _SC_ROWS // _NW // _CHUNK
_ROWS_PER_W = _SC_ROWS // _NW

_TC_R = 4096
_TC_NB = (_N - _SC_ROWS) // _TC_R
_TC_BLK0 = _SC_ROWS // _TC_R
assert _SC_ROWS + _TC_NB * _TC_R == _N and _SC_ROWS % _TC_R == 0

def _sc_body(sm_hbm, lbl_hbm, out_hbm,
             rows0, rows1, lbl0, lbl1, acc,
             sem_r0, sem_r1, sem_l0, sem_l1):
    wid = lax.axis_index("s") * _NC + lax.axis_index("c")
    base = wid * _ROWS_PER_W

    rows_bufs = (rows0, rows1)
    lbl_bufs = (lbl0, lbl1)
    sem_r = (sem_r0, sem_r1)
    sem_l = (sem_l0, sem_l1)

    lanes = lax.iota(jnp.int32, 16)
    zero16 = jnp.zeros((16,), jnp.float32)

    for t in range(48):
        acc[pl.ds(t * 16, 16)] = zero16

    def start(cur, b):
        r0 = base + cur * _CHUNK
        pltpu.async_copy(sm_hbm.at[pl.ds(r0 * _C, _CHUNK * _C)], rows_bufs[b],
                         sem_r[b])
        pltpu.async_copy(lbl_hbm.at[pl.ds(r0, _CHUNK)], lbl_bufs[b], sem_l[b])

    def wait(cur, b):
        r0 = base + cur * _CHUNK
        pltpu.make_async_copy(
            sm_hbm.at[pl.ds(r0 * _C, _CHUNK * _C)], rows_bufs[b], sem_r[b]).wait()
        pltpu.make_async_copy(
            lbl_hbm.at[pl.ds(r0, _CHUNK)], lbl_bufs[b], sem_l[b]).wait()

    nblk = _C // 16
    idx_pat = [lanes * _C + ((lanes + u) & 15) for u in range(16)]

    def compute(b):
        rows = rows_bufs[b]
        lblv = lbl_bufs[b]

        def gbody(g, _):
            gbase = g * (16 * _C)

            m_blks = []
            for bb in range(nblk):
                slb = rows.at[pl.ds(gbase + bb * 16, 15 * _C + 16)]
                mb = jnp.full((16,), -1.0, jnp.float32)
                for u in range(16):
                    v = plsc.load_gather(slb, [idx_pat[u]])
                    mb = jnp.maximum(mb, v)
                m_blks.append(mb)

            t1 = [jnp.maximum(m_blks[2 * i], m_blks[2 * i + 1]) for i in range(4)]
            t2 = [jnp.maximum(t1[0], t1[1]), jnp.maximum(t1[2], t1[3])]
            m = jnp.maximum(t2[0], t2[1])

            amb = jnp.zeros((16,), jnp.int32)
            for bb in range(nblk - 1, 0, -1):
                amb = jnp.where(m_blks[bb] == m,
                                jnp.full((16,), bb * 16, jnp.int32), amb)

            sl = rows.at[pl.ds(gbase, 16 * _C)]
            am = amb
            for u in range(15, -1, -1):
                idxfull = idx_pat[u] + amb
                v = plsc.load_gather(sl, [idxfull])
                am = jnp.where(v == m, idxfull & 127, am)

            lbl = lblv[pl.ds(g * 16, 16)]
            accv = jnp.where(am == lbl, 1.0, 0.0).astype(jnp.float32)

            bini = (m * jnp.float32(_NBINS)).astype(jnp.int32)
            binv = jnp.minimum(bini, _NBINS - 1)
            validf = jnp.where(m > 0.0, 1.0, 0.0).astype(jnp.float32)

            slot = binv * 16 + lanes
            plsc.addupdate_scatter(acc, [slot], validf)
            plsc.addupdate_scatter(acc, [slot + 256], m * validf)
            plsc.addupdate_scatter(acc, [slot + 512], accv * validf)
            return 0

        lax.fori_loop(0, _GROUPS, gbody, 0)

    start(0, 0)
    start(1, 1)

    def outer(it, _):
        i = it * 2
        for b in range(2):
            cur = i + b
            wait(cur, b)
            compute(b)

            @pl.when(cur + 2 < _SC_NCHUNK)
            def _prefetch():
                start(cur + 2, b)

        return 0

    lax.fori_loop(0, _SC_NCHUNK // 2, outer, 0)

    pltpu.sync_copy(acc, out_hbm.at[wid])


_sc_partials = functools.partial(
    pl.kernel,
    out_type=jax.ShapeDtypeStruct((_NW, 768), jnp.float32),
    mesh=plsc.VectorSubcoreMesh(core_axis_name="c", subcore_axis_name="s"),
    compiler_params=pltpu.CompilerParams(needs_layout_passes=False),
    scratch_types=[
        pltpu.VMEM((_CHUNK * _C,), jnp.float32),
        pltpu.VMEM((_CHUNK * _C,), jnp.float32),
        pltpu.VMEM((_CHUNK,), jnp.int32),
        pltpu.VMEM((_CHUNK,), jnp.int32),
        pltpu.VMEM((768,), jnp.float32),
        pltpu.SemaphoreType.DMA,
        pltpu.SemaphoreType.DMA,
        pltpu.SemaphoreType.DMA,
        pltpu.SemaphoreType.DMA,
    ],
)(_sc_body)


def _tc_body(lbl_ref, sm_ref, out_ref):
    i = pl.program_id(0)

    @pl.when(i == 0)
    def _init():
        out_ref[...] = jnp.zeros((8, 128), jnp.float32)

    xt = sm_ref[...].T
    conf = jnp.max(xt, axis=0, keepdims=True)
    iot = lax.broadcasted_iota(jnp.int32, (_C, _TC_R), 0)
    cand = jnp.where(xt == conf, iot, _C)
    pred = jnp.min(cand, axis=0, keepdims=True)

    lbl = lbl_ref[0]
    accv = jnp.where(pred == lbl, 1.0, 0.0).astype(jnp.float32)

    bini = (conf * jnp.float32(_NBINS)).astype(jnp.int32)
    t = jnp.where(conf > 0.0, jnp.minimum(bini, _NBINS - 1) + 1, 0)

    onehot = jnp.where(
        lax.broadcasted_iota(jnp.int32, (16, _TC_R), 0) == t, 1.0, 0.0
    ).astype(jnp.float32)
    vals = jnp.concatenate(
        [jnp.ones((1, _TC_R), jnp.float32), conf, accv], axis=0)
    stats = lax.dot_general(vals, onehot, (((1,), (1,)), ((), ())),
                            preferred_element_type=jnp.float32)
    out_ref[0:3, 0:16] = out_ref[0:3, 0:16] + stats


_tc_stats = pl.pallas_call(
    _tc_body,
    grid=(_TC_NB,),
    in_specs=[
        pl.BlockSpec((1, 1, _TC_R), lambda i: (_TC_BLK0 + i, 0, 0)),
        pl.BlockSpec((_TC_R, _C), lambda i: (_TC_BLK0 + i, 0)),
    ],
    out_specs=pl.BlockSpec((8, 128), lambda i: (0, 0)),
    out_shape=jax.ShapeDtypeStruct((8, 128), jnp.float32),
)


def kernel(softmaxes, labels):
    parts_sc = _sc_partials(softmaxes.reshape(-1), labels)
    parts_tc = _tc_stats(labels.reshape(-1, 1, _TC_R), softmaxes)
    s = jnp.sum(parts_sc, axis=0).reshape(48, 16).sum(axis=1)
    cnt = s[0:_NBINS] + parts_tc[0, 1:16]
    sum_conf = s[16:16 + _NBINS] + parts_tc[1, 1:16]
    sum_acc = s[32:32 + _NBINS] + parts_tc[2, 1:16]
    prop = cnt / _N
    safe = jnp.maximum(cnt, 1.0)
    contrib = jnp.abs(sum_conf / safe - sum_acc / safe) * prop
    ece = jnp.sum(jnp.where(prop > 0.0, contrib, 0.0))
    return ece.reshape(1)
